# baseline (speedup 1.0000x reference)
import jax
import jax.numpy as jnp
from jax.experimental import pallas as pl
from jax.experimental.pallas import tpu as pltpu

_NCOPIES = 8


def kernel(x, emb_weight):
    sl = x.shape[1]
    dim = emb_weight.shape[1]
    slab = sl // _NCOPIES

    def dma_body(in_hbm, out_hbm, *sems):
        def copy(i):
            return pltpu.make_async_copy(
                in_hbm.at[pl.ds(i * slab, slab)],
                out_hbm.at[pl.ds(i * slab, slab)],
                sems[i],
            )

        for i in range(_NCOPIES):
            copy(i).start()
        for i in range(_NCOPIES):
            copy(i).wait()

    return pl.pallas_call(
        dma_body,
        out_shape=jax.ShapeDtypeStruct((sl, dim), emb_weight.dtype),
        in_specs=[pl.BlockSpec(memory_space=pl.ANY)],
        out_specs=pl.BlockSpec(memory_space=pl.ANY),
        scratch_shapes=[pltpu.SemaphoreType.DMA] * _NCOPIES,
    )(emb_weight)


# speedup vs baseline: 20.2370x; 20.2370x over previous
import functools

import jax
import jax.numpy as jnp
from jax import lax
from jax.experimental import pallas as pl
from jax.experimental.pallas import tpu as pltpu
from jax.experimental.pallas import tpu_sc as plsc

_TC_ROWS = 5120
_TC_BLOCK_ROWS = 1024
_CHUNK_ROWS = 16
_NBUF = 2


def _tc_body(in_ref, out_ref):
    out_ref[...] = in_ref[...]


def _tc_copy(emb_weight, rows, dim):
    return pl.pallas_call(
        _tc_body,
        out_shape=jax.ShapeDtypeStruct((rows, dim), emb_weight.dtype),
        grid=(rows // _TC_BLOCK_ROWS,),
        in_specs=[pl.BlockSpec((_TC_BLOCK_ROWS, dim), lambda i: (i, 0))],
        out_specs=pl.BlockSpec((_TC_BLOCK_ROWS, dim), lambda i: (i, 0)),
    )(emb_weight)


def _sc_copy(emb_weight, row0, rows, dim):
    info = plsc.get_sparse_core_info()
    nw = info.num_cores * info.num_subcores
    rows_per_w = rows // nw
    nchunks = rows_per_w // _CHUNK_ROWS
    mesh = plsc.VectorSubcoreMesh(core_axis_name="c", subcore_axis_name="s")

    @functools.partial(
        pl.kernel,
        mesh=mesh,
        out_type=jax.ShapeDtypeStruct((rows, dim), emb_weight.dtype),
        scratch_types=(
            [pltpu.VMEM((_NBUF, _CHUNK_ROWS, dim), emb_weight.dtype)]
            + [pltpu.SemaphoreType.DMA] * (2 * _NBUF)
        ),
    )
    def copy_kernel(table_hbm, out_hbm, buf, *sems):
        wid = lax.axis_index("s") * info.num_cores + lax.axis_index("c")
        base = wid * rows_per_w
        in_sems = sems[:_NBUF]
        out_sems = sems[_NBUF:]

        def in_copy(c, b):
            return pltpu.make_async_copy(
                table_hbm.at[pl.ds(row0 + base + c * _CHUNK_ROWS, _CHUNK_ROWS)],
                buf.at[b],
                in_sems[b],
            )

        def out_copy(c, b):
            return pltpu.make_async_copy(
                buf.at[b],
                out_hbm.at[pl.ds(base + c * _CHUNK_ROWS, _CHUNK_ROWS)],
                out_sems[b],
            )

        for b in range(_NBUF):
            in_copy(b, b).start()
        for c in range(nchunks):
            b = c % _NBUF
            in_copy(c, b).wait()
            out_copy(c, b).start()
            nxt = c + _NBUF
            if nxt < nchunks:
                out_copy(c, b).wait()
                in_copy(nxt, b).start()
        for c in range(nchunks - _NBUF, nchunks):
            out_copy(c, c % _NBUF).wait()

    return copy_kernel(emb_weight)


def kernel(x, emb_weight):
    sl = x.shape[1]
    dim = emb_weight.shape[1]
    tc_out = _tc_copy(emb_weight, _TC_ROWS, dim)
    sc_out = _sc_copy(emb_weight, _TC_ROWS, sl - _TC_ROWS, dim)
    return jnp.concatenate([tc_out, sc_out], axis=0)


# speedup vs baseline: 27.3821x; 1.3531x over previous
import functools

import jax
import jax.numpy as jnp
from jax import lax
from jax.experimental import pallas as pl
from jax.experimental.pallas import tpu as pltpu
from jax.experimental.pallas import tpu_sc as plsc

_CHUNK_ROWS = 256
_NBUF = 2


def kernel(x, emb_weight):
    sl = x.shape[1]
    dim = emb_weight.shape[1]
    info = plsc.get_sparse_core_info()
    nc = info.num_cores
    rows_per_c = sl // nc
    nchunks = rows_per_c // _CHUNK_ROWS
    mesh = plsc.VectorSubcoreMesh(core_axis_name="c", subcore_axis_name="s")

    @functools.partial(
        pl.kernel,
        mesh=mesh,
        out_type=jax.ShapeDtypeStruct((sl, dim), emb_weight.dtype),
        scratch_types=(
            [pltpu.VMEM_SHARED((_NBUF, _CHUNK_ROWS, dim), emb_weight.dtype)]
            + [pltpu.SemaphoreType.DMA] * (2 * _NBUF)
        ),
    )
    def copy_kernel(table_hbm, out_hbm, buf, *sems):
        cid = lax.axis_index("c")
        sid = lax.axis_index("s")
        base = cid * rows_per_c
        in_sems = sems[:_NBUF]
        out_sems = sems[_NBUF:]

        def in_copy(c, b):
            return pltpu.make_async_copy(
                table_hbm.at[pl.ds(base + c * _CHUNK_ROWS, _CHUNK_ROWS)],
                buf.at[b],
                in_sems[b],
            )

        def out_copy(c, b):
            return pltpu.make_async_copy(
                buf.at[b],
                out_hbm.at[pl.ds(base + c * _CHUNK_ROWS, _CHUNK_ROWS)],
                out_sems[b],
            )

        @pl.when(sid == 0)
        def _():
            for b in range(_NBUF):
                in_copy(b, b).start()
            for c in range(nchunks):
                b = c % _NBUF
                in_copy(c, b).wait()
                out_copy(c, b).start()
                nxt = c + _NBUF
                if nxt < nchunks:
                    out_copy(c, b).wait()
                    in_copy(nxt, b).start()
            for c in range(nchunks - _NBUF, nchunks):
                out_copy(c, c % _NBUF).wait()

    return copy_kernel(emb_weight)


# speedup vs baseline: 30.3495x; 1.1084x over previous
import functools

import jax
import jax.numpy as jnp
from jax import lax
from jax.experimental import pallas as pl
from jax.experimental.pallas import tpu as pltpu
from jax.experimental.pallas import tpu_sc as plsc

_ST_CHUNK = 16
_ST_NCHUNKS = 8
_ST_ROWS = _ST_CHUNK * _ST_NCHUNKS
_N_STREAM_TILES = 15
_DMA_NBUF = 2
_NBUF = 2


def kernel(x, emb_weight):
    sl = x.shape[1]
    dim = emb_weight.shape[1]
    info = plsc.get_sparse_core_info()
    nc = info.num_cores
    ns = info.num_subcores
    rows_per_c = sl // nc
    st_rows_per_c = _N_STREAM_TILES * _ST_ROWS
    dma_rows = rows_per_c - st_rows_per_c
    dma_chunk = 128
    dma_nchunks = dma_rows // dma_chunk
    mesh = plsc.VectorSubcoreMesh(core_axis_name="c", subcore_axis_name="s")

    @functools.partial(
        pl.kernel,
        mesh=mesh,
        out_type=jax.ShapeDtypeStruct((sl, dim), emb_weight.dtype),
        scratch_types=(
            [pltpu.VMEM((_NBUF, _ST_CHUNK, dim), emb_weight.dtype)]
            + [pltpu.VMEM_SHARED((_DMA_NBUF, dma_chunk, dim), emb_weight.dtype)]
            + [pltpu.SemaphoreType.DMA] * (2 * _NBUF)
        ),
    )
    def copy_kernel(table_hbm, out_hbm, st_buf, dma_buf, *sems):
        cid = lax.axis_index("c")
        sid = lax.axis_index("s")
        c_base = cid * rows_per_c
        in_sems = sems[:_NBUF]
        out_sems = sems[_NBUF:]

        st_base = c_base + sid * _ST_ROWS

        def st_in(c, b):
            return pltpu.make_async_copy(
                table_hbm.at[pl.ds(st_base + c * _ST_CHUNK, _ST_CHUNK)],
                st_buf.at[b],
                in_sems[b],
            )

        def st_out(c, b):
            return pltpu.make_async_copy(
                st_buf.at[b],
                out_hbm.at[pl.ds(st_base + c * _ST_CHUNK, _ST_CHUNK)],
                out_sems[b],
            )

        @pl.when(sid < _N_STREAM_TILES)
        def _():
            for b in range(_NBUF):
                st_in(b, b).start()
            for c in range(_ST_NCHUNKS):
                b = c % _NBUF
                st_in(c, b).wait()
                st_out(c, b).start()
                nxt = c + _NBUF
                if nxt < _ST_NCHUNKS:
                    st_out(c, b).wait()
                    st_in(nxt, b).start()
            for c in range(_ST_NCHUNKS - _NBUF, _ST_NCHUNKS):
                st_out(c, c % _NBUF).wait()

        dma_base = c_base + st_rows_per_c

        def dma_in(c, b):
            return pltpu.make_async_copy(
                table_hbm.at[pl.ds(dma_base + c * dma_chunk, dma_chunk)],
                dma_buf.at[b],
                in_sems[b],
            )

        def dma_out(c, b):
            return pltpu.make_async_copy(
                dma_buf.at[b],
                out_hbm.at[pl.ds(dma_base + c * dma_chunk, dma_chunk)],
                out_sems[b],
            )

        @pl.when(sid == ns - 1)
        def _():
            for b in range(_DMA_NBUF):
                dma_in(b, b).start()
            for c in range(dma_nchunks):
                b = c % _DMA_NBUF
                dma_in(c, b).wait()
                dma_out(c, b).start()
                nxt = c + _DMA_NBUF
                if nxt < dma_nchunks:
                    dma_out(c, b).wait()
                    dma_in(nxt, b).start()
            for c in range(dma_nchunks - _DMA_NBUF, dma_nchunks):
                dma_out(c, c % _DMA_NBUF).wait()

    return copy_kernel(emb_weight)


# speedup vs baseline: 31.3780x; 1.0339x over previous
import functools

import jax
import jax.numpy as jnp
from jax import lax
from jax.experimental import pallas as pl
from jax.experimental.pallas import tpu as pltpu
from jax.experimental.pallas import tpu_sc as plsc

_CHUNK_ROWS = 16
_NBUF = 3


def kernel(x, emb_weight):
    sl = x.shape[1]
    dim = emb_weight.shape[1]
    info = plsc.get_sparse_core_info()
    nw = info.num_cores * info.num_subcores
    rows_per_w = sl // nw
    nchunks = rows_per_w // _CHUNK_ROWS
    mesh = plsc.VectorSubcoreMesh(core_axis_name="c", subcore_axis_name="s")

    @functools.partial(
        pl.kernel,
        mesh=mesh,
        out_type=jax.ShapeDtypeStruct((sl, dim), emb_weight.dtype),
        scratch_types=(
            [pltpu.VMEM((_NBUF, _CHUNK_ROWS, dim), emb_weight.dtype)]
            + [pltpu.SemaphoreType.DMA] * (2 * _NBUF)
        ),
    )
    def copy_kernel(table_hbm, out_hbm, buf, *sems):
        wid = lax.axis_index("s") * info.num_cores + lax.axis_index("c")
        base = wid * rows_per_w
        in_sems = sems[:_NBUF]
        out_sems = sems[_NBUF:]

        def in_copy(c, b):
            return pltpu.make_async_copy(
                table_hbm.at[pl.ds(base + c * _CHUNK_ROWS, _CHUNK_ROWS)],
                buf.at[b],
                in_sems[b],
            )

        def out_copy(c, b):
            return pltpu.make_async_copy(
                buf.at[b],
                out_hbm.at[pl.ds(base + c * _CHUNK_ROWS, _CHUNK_ROWS)],
                out_sems[b],
            )

        for b in range(_NBUF):
            in_copy(b, b).start()
        for c in range(nchunks):
            b = c % _NBUF
            in_copy(c, b).wait()
            out_copy(c, b).start()
            nxt = c + _NBUF
            if nxt < nchunks:
                out_copy(c, b).wait()
                in_copy(nxt, b).start()
        for c in range(nchunks - _NBUF, nchunks):
            out_copy(c, c % _NBUF).wait()

    return copy_kernel(emb_weight)


# speedup vs baseline: 31.4727x; 1.0030x over previous
import functools

import jax
from jax import lax
from jax.experimental import pallas as pl
from jax.experimental.pallas import tpu as pltpu
from jax.experimental.pallas import tpu_sc as plsc

_CHUNK_ROWS = 16
_NBUF = 3


def kernel(x, emb_weight):
    sl = x.shape[1]
    dim = emb_weight.shape[1]
    info = plsc.get_sparse_core_info()
    nw = info.num_cores * info.num_subcores
    rows_per_w = sl // nw
    nchunks = rows_per_w // _CHUNK_ROWS
    mesh = plsc.VectorSubcoreMesh(core_axis_name="c", subcore_axis_name="s")

    @functools.partial(
        pl.kernel,
        mesh=mesh,
        out_type=jax.ShapeDtypeStruct((sl, dim), emb_weight.dtype),
        scratch_types=(
            [pltpu.VMEM((_NBUF, _CHUNK_ROWS, dim), emb_weight.dtype)]
            + [pltpu.SemaphoreType.DMA] * (2 * _NBUF)
        ),
    )
    def copy_kernel(table_hbm, out_hbm, buf, *sems):
        wid = lax.axis_index("s") * info.num_cores + lax.axis_index("c")
        base = wid * rows_per_w
        in_sems = sems[:_NBUF]
        out_sems = sems[_NBUF:]

        def in_copy(c, b):
            return pltpu.make_async_copy(
                table_hbm.at[pl.ds(base + c * _CHUNK_ROWS, _CHUNK_ROWS)],
                buf.at[b],
                in_sems[b],
            )

        def out_copy(c, b):
            return pltpu.make_async_copy(
                buf.at[b],
                out_hbm.at[pl.ds(base + c * _CHUNK_ROWS, _CHUNK_ROWS)],
                out_sems[b],
            )

        for b in range(_NBUF):
            in_copy(b, b).start()
        for c in range(nchunks):
            b = c % _NBUF
            in_copy(c, b).wait()
            out_copy(c, b).start()
            nxt = c + _NBUF
            if nxt < nchunks:
                out_copy(c, b).wait()
                in_copy(nxt, b).start()
        for c in range(nchunks - _NBUF, nchunks):
            out_copy(c, c % _NBUF).wait()

    return copy_kernel(emb_weight)
